# baseline (speedup 1.0000x reference)
import jax
import jax.numpy as jnp
from jax import lax
from jax.experimental import pallas as pl
from jax.experimental.pallas import tpu as pltpu
from jax.experimental.pallas import tpu_sc as plsc

_NUM_CLASSES = 100000
_FEAT = 64
_BATCH = 16384
_NC = 2
_NS = 16
_NW = _NC * _NS
_BPW = _BATCH // _NW
_CHUNK = 128
_NCHUNK = _BPW // _CHUNK
_GRP = _CHUNK // 16
_UNROLL = 8


def _cl_kernel(feat_hbm, lab_hbm, cent_hbm, out_hbm,
               lab_v, idx_v, rows_v, feat_v, acc_v, gsem, fsem):
    wid = lax.axis_index("c") * _NS + lax.axis_index("s")
    base = wid * _BPW
    iota = lax.iota(jnp.int32, 16)

    pltpu.sync_copy(lab_hbm.at[pl.ds(wid * _NCHUNK, _NCHUNK)], lab_v)

    def stage(j, buf):
        for g in range(_GRP):
            sl = pl.ds(16 * g, 16)
            idx_v[j, sl] = lab_v[j, sl] >> 1
        gat = pltpu.async_copy(cent_hbm.at[idx_v.at[j]], rows_v.at[j % 2],
                               gsem.at[j % 2])
        fc = pltpu.async_copy(feat_hbm.at[pl.ds(base + j * _CHUNK, _CHUNK)],
                              feat_v.at[j % 2], fsem.at[j % 2])
        return gat, fc

    zeros = jnp.zeros((16,), jnp.float32)
    accs = (zeros, zeros, zeros, zeros)
    pend = stage(0, 0)
    for j in range(_NCHUNK):
        pend[0].wait()
        pend[1].wait()
        if j + 1 < _NCHUNK:
            pend = stage(j + 1, (j + 1) % 2)
        rows = rows_v.at[j % 2]
        feat = feat_v.at[j % 2]
        for g in range(_GRP):
            sl = pl.ds(16 * g, 16)
            labv = lab_v[j, sl]
            colbase = (labv & 1) * _FEAT
            rowi = iota + (16 * g)

            def dbody(d, accs, _rows=rows, _feat=feat,
                      _rowi=rowi, _colbase=colbase):
                a0, a1, a2, a3 = accs
                for u in range(_UNROLL):
                    dcol = d + u
                    dvec = jnp.full((16,), 0, jnp.int32) + dcol
                    c = plsc.load_gather(_rows, [_rowi, _colbase + dcol])
                    f = plsc.load_gather(_feat, [_rowi, dvec])
                    diff = f - c
                    if u % 4 == 0:
                        a0 = a0 + diff * diff
                    elif u % 4 == 1:
                        a1 = a1 + diff * diff
                    elif u % 4 == 2:
                        a2 = a2 + diff * diff
                    else:
                        a3 = a3 + diff * diff
                return (a0, a1, a2, a3)

            accs = lax.fori_loop(0, _FEAT // _UNROLL, lambda i, a: dbody(
                i * _UNROLL, a), accs, unroll=1)

    total = (accs[0] + accs[1]) + (accs[2] + accs[3])
    acc_v[...] = total * jnp.float32(1.0 / _BATCH)
    pltpu.sync_copy(acc_v, out_hbm.at[wid])


@jax.jit
def _center_loss(features, labels, centers):
    labels2 = labels.reshape(_BATCH // _CHUNK, _CHUNK)
    centers2 = centers.reshape(_NUM_CLASSES // 2, 2 * _FEAT)
    mesh = plsc.VectorSubcoreMesh(
        core_axis_name="c", subcore_axis_name="s",
        num_cores=_NC, num_subcores=_NS)
    out = pl.kernel(
        _cl_kernel,
        out_type=jax.ShapeDtypeStruct((_NW, 16), jnp.float32),
        mesh=mesh,
        compiler_params=pltpu.CompilerParams(needs_layout_passes=False),
        scratch_types=[
            pltpu.VMEM((_NCHUNK, _CHUNK), jnp.int32),
            pltpu.VMEM((_NCHUNK, _CHUNK), jnp.int32),
            pltpu.VMEM((2, _CHUNK, 2 * _FEAT), jnp.float32),
            pltpu.VMEM((2, _CHUNK, _FEAT), jnp.float32),
            pltpu.VMEM((16,), jnp.float32),
            pltpu.SemaphoreType.DMA((2,)),
            pltpu.SemaphoreType.DMA((2,)),
        ],
    )(features, labels2, centers2)
    return jnp.sum(out)


def kernel(features, labels, centers):
    return _center_loss(features, labels.astype(jnp.int32), centers)


# speedup vs baseline: 1.7078x; 1.7078x over previous
import jax
import jax.numpy as jnp
from jax import lax
from jax.experimental import pallas as pl
from jax.experimental.pallas import tpu as pltpu
from jax.experimental.pallas import tpu_sc as plsc

_NUM_CLASSES = 100000
_FEAT = 64
_BATCH = 16384
_NC = 2
_NS = 16
_NW = _NC * _NS
_BPW = _BATCH // _NW
_CHUNK = 128
_NCHUNK = _BPW // _CHUNK


def _cl_kernel(feat_hbm, lab_hbm, cent_hbm, out_hbm,
               lab_v, rows_v, feat_v, acc_v, gsem, fsem):
    wid = lax.axis_index("c") * _NS + lax.axis_index("s")
    base = wid * _BPW

    pltpu.sync_copy(lab_hbm.at[pl.ds(wid * _NCHUNK, _NCHUNK)], lab_v)

    def stage(j):
        buf = rows_v.at[j % 2]

        def issue(g, _):
            labv = lab_v[j, pl.ds(g * 16, 16)]
            for lane in range(16):
                l = labv[lane]
                pltpu.async_copy(cent_hbm.at[pl.ds(l, 1)],
                                 buf.at[pl.ds(g * 16 + lane, 1)],
                                 gsem.at[j % 2])
            return 0

        lax.fori_loop(0, _CHUNK // 16, issue, 0)
        fc = pltpu.async_copy(feat_hbm.at[pl.ds(base + j * _CHUNK, _CHUNK)],
                              feat_v.at[j % 2], fsem.at[j % 2])
        return fc

    def drain(j):
        pltpu.make_async_copy(cent_hbm.at[pl.ds(0, _CHUNK)],
                              rows_v.at[j % 2], gsem.at[j % 2]).wait()

    zeros = jnp.zeros((16,), jnp.float32)
    accs = (zeros, zeros, zeros, zeros)
    pend = stage(0)
    for j in range(_NCHUNK):
        pend.wait()
        drain(j)
        if j + 1 < _NCHUNK:
            pend = stage(j + 1)
        rows = rows_v.at[j % 2]
        feat = feat_v.at[j % 2]

        def row_body(r, accs, _rows=rows, _feat=feat):
            a0, a1, a2, a3 = accs
            f0 = _feat[r, pl.ds(0, 16)]
            c0 = _rows[r, pl.ds(0, 16)]
            d0 = f0 - c0
            a0 = a0 + d0 * d0
            f1 = _feat[r, pl.ds(16, 16)]
            c1 = _rows[r, pl.ds(16, 16)]
            d1 = f1 - c1
            a1 = a1 + d1 * d1
            f2 = _feat[r, pl.ds(32, 16)]
            c2 = _rows[r, pl.ds(32, 16)]
            d2 = f2 - c2
            a2 = a2 + d2 * d2
            f3 = _feat[r, pl.ds(48, 16)]
            c3 = _rows[r, pl.ds(48, 16)]
            d3 = f3 - c3
            a3 = a3 + d3 * d3
            return (a0, a1, a2, a3)

        accs = lax.fori_loop(0, _CHUNK, row_body, accs)

    total = (accs[0] + accs[1]) + (accs[2] + accs[3])
    acc_v[...] = total * jnp.float32(1.0 / _BATCH)
    pltpu.sync_copy(acc_v, out_hbm.at[wid])


@jax.jit
def _center_loss(features, labels, centers):
    labels2 = labels.reshape(_BATCH // _CHUNK, _CHUNK)
    mesh = plsc.VectorSubcoreMesh(
        core_axis_name="c", subcore_axis_name="s",
        num_cores=_NC, num_subcores=_NS)
    out = pl.kernel(
        _cl_kernel,
        out_type=jax.ShapeDtypeStruct((_NW, 16), jnp.float32),
        mesh=mesh,
        scratch_types=[
            pltpu.VMEM((_NCHUNK, _CHUNK), jnp.int32),
            pltpu.VMEM((2, _CHUNK, _FEAT), jnp.float32),
            pltpu.VMEM((2, _CHUNK, _FEAT), jnp.float32),
            pltpu.VMEM((16,), jnp.float32),
            pltpu.SemaphoreType.DMA((2,)),
            pltpu.SemaphoreType.DMA((2,)),
        ],
    )(features, labels2, centers)
    return jnp.sum(out)


def kernel(features, labels, centers):
    return _center_loss(features, labels.astype(jnp.int32), centers)
